# baseline (speedup 1.0000x reference)
import functools

import jax
import jax.numpy as jnp
from jax import lax
from jax.experimental import pallas as pl
from jax.experimental.pallas import tpu as pltpu
from jax.experimental.pallas import tpu_sc as plsc

N = 10000
E = 320000
D = 128
EH = 64

NC = 2
NS = 16
EPC = E // NC
EPT = EPC // NS
CH = 80
NCHUNK = EPT // CH

ROWS_A = 640
ROWS_B = N - 15 * ROWS_A


NPAIR = NCHUNK // 2


DEG_R = 4
AGG_R = 2


def _deg_body(dst3_hbm, zeros_hbm, out_hbm, didx_v, ones_v, acc, *sems):
    c = lax.axis_index("c")
    s = lax.axis_index("s")
    w = c * NS + s

    @pl.when(s == 0)
    def _():
        pltpu.sync_copy(zeros_hbm, acc)

    pltpu.sync_copy(dst3_hbm.at[w], didx_v)
    for i in range(CH // 16):
        ones_v[pl.ds(i * 16, 16)] = jnp.ones((16,), jnp.float32)
    plsc.subcore_barrier()

    def scat(ci, k):
        pltpu.async_copy(ones_v, acc.at[didx_v.at[ci]], sems[k], add=True)

    def wait(ci, k):
        pltpu.make_async_copy(ones_v, acc.at[didx_v.at[ci]], sems[k]).wait()

    for k in range(DEG_R):
        scat(k, k)

    nfull = (NCHUNK - 1) // DEG_R

    def body(i, _):
        for k in range(DEG_R):
            ci = i * DEG_R + k
            wait(ci, k)

            @pl.when(ci + DEG_R < NCHUNK)
            def _():
                scat(ci + DEG_R, k)

        return 0

    lax.fori_loop(0, nfull, body, 0)
    for k in range(NCHUNK - nfull * DEG_R):
        wait(nfull * DEG_R + k, k)
    plsc.subcore_barrier()

    @pl.when(s == 0)
    def _():
        pltpu.sync_copy(acc, out_hbm.at[c])


def _agg_body(y_hbm, src_hbm, dst3_hbm, zeros_hbm, out_hbm,
              sidx_v, didx_v, *rest):
    rows = rest[:AGG_R]
    acc = rest[AGG_R]
    gsem = rest[AGG_R + 1:AGG_R + 1 + AGG_R]
    ssem = rest[AGG_R + 1 + AGG_R:]
    c = lax.axis_index("c")
    s = lax.axis_index("s")
    w = c * NS + s

    @pl.when(s < 15)
    def _():
        pltpu.sync_copy(zeros_hbm.at[pl.ds(s * ROWS_A, ROWS_A)],
                        acc.at[pl.ds(s * ROWS_A, ROWS_A)])

    @pl.when(s == 15)
    def _():
        pltpu.sync_copy(zeros_hbm.at[pl.ds(15 * ROWS_A, ROWS_B)],
                        acc.at[pl.ds(15 * ROWS_A, ROWS_B)])

    pltpu.sync_copy(src_hbm.at[pl.ds(w * EPT, EPT)], sidx_v)
    pltpu.sync_copy(dst3_hbm.at[w], didx_v)
    plsc.subcore_barrier()

    def gat(ci, k):
        pltpu.async_copy(y_hbm.at[sidx_v.at[pl.ds(ci * CH, CH)]], rows[k],
                         gsem[k])

    def wait_gat(ci, k):
        pltpu.make_async_copy(y_hbm.at[sidx_v.at[pl.ds(ci * CH, CH)]],
                              rows[k], gsem[k]).wait()

    def scat(ci, k):
        pltpu.async_copy(rows[k], acc.at[didx_v.at[ci]], ssem[k], add=True)

    def wait_scat(ci, k):
        pltpu.make_async_copy(rows[k], acc.at[didx_v.at[ci]], ssem[k]).wait()

    for k in range(AGG_R):
        gat(k, k)

    nfull = NCHUNK // AGG_R

    def body(i, _):
        for k in range(AGG_R):
            ci = i * AGG_R + k
            wait_gat(ci, k)
            scat(ci, k)
        for k in range(AGG_R):
            ci = i * AGG_R + k
            wait_scat(ci, k)

            @pl.when(ci + AGG_R < NCHUNK)
            def _():
                gat(ci + AGG_R, k)

        return 0

    lax.fori_loop(0, nfull, body, 0)
    for k in range(NCHUNK - nfull * AGG_R):
        ci = nfull * AGG_R + k
        wait_gat(ci, k)
        scat(ci, k)
    for k in range(NCHUNK - nfull * AGG_R):
        wait_scat(nfull * AGG_R + k, k)
    plsc.subcore_barrier()

    @pl.when(s < 15)
    def _():
        pltpu.sync_copy(acc.at[pl.ds(s * ROWS_A, ROWS_A)],
                        out_hbm.at[c, pl.ds(s * ROWS_A, ROWS_A)])

    @pl.when(s == 15)
    def _():
        pltpu.sync_copy(acc.at[pl.ds(15 * ROWS_A, ROWS_B)],
                        out_hbm.at[c, pl.ds(15 * ROWS_A, ROWS_B)])


EHALF = E // 2
RPT = EHALF // (NC * NS)
CH2 = 80
NFULL2 = RPT // CH2
TAIL2 = RPT - NFULL2 * CH2


PAIR_R = 3


def _pair_body(a_hbm, b_hbm, src_hbm, dst_hbm, out_hbm,
               sl_v, dl_v, sh_v, dh_v, *rest):
    gbufs = [rest[4 * k:4 * k + 4] for k in range(PAIR_R)]
    sbufs = rest[4 * PAIR_R:5 * PAIR_R]
    gsem = rest[5 * PAIR_R:6 * PAIR_R]
    wsem = rest[6 * PAIR_R:7 * PAIR_R]
    c = lax.axis_index("c")
    s = lax.axis_index("s")
    w = c * NS + s
    rbase = w * RPT

    pltpu.sync_copy(src_hbm.at[pl.ds(rbase, RPT)], sl_v)
    pltpu.sync_copy(dst_hbm.at[pl.ds(rbase, RPT)], dl_v)
    pltpu.sync_copy(src_hbm.at[pl.ds(EHALF + rbase, RPT)], sh_v)
    pltpu.sync_copy(dst_hbm.at[pl.ds(EHALF + rbase, RPT)], dh_v)

    def bufs(k, n):
        gs, gd, gsh, gdh = gbufs[k]
        if n == CH2:
            return gs, gd, gsh, gdh, sbufs[k]
        sl = pl.ds(0, n)
        return (gs.at[sl], gd.at[sl], gsh.at[sl], gdh.at[sl],
                sbufs[k].at[sl])

    def gat(ci, k, n=CH2):
        gs, gd, gsh, gdh, _ = bufs(k, n)
        off = ci * CH2
        pltpu.async_copy(a_hbm.at[sl_v.at[pl.ds(off, n)]], gs, gsem[k])
        pltpu.async_copy(b_hbm.at[dl_v.at[pl.ds(off, n)]], gd, gsem[k])
        pltpu.async_copy(a_hbm.at[sh_v.at[pl.ds(off, n)]], gsh, gsem[k])
        pltpu.async_copy(b_hbm.at[dh_v.at[pl.ds(off, n)]], gdh, gsem[k])

    def wait_gat(ci, k, n=CH2):
        gs, gd, gsh, gdh, _ = bufs(k, n)
        off = ci * CH2
        pltpu.make_async_copy(
            a_hbm.at[sl_v.at[pl.ds(off, n)]], gs, gsem[k]).wait()
        pltpu.make_async_copy(
            b_hbm.at[dl_v.at[pl.ds(off, n)]], gd, gsem[k]).wait()
        pltpu.make_async_copy(
            a_hbm.at[sh_v.at[pl.ds(off, n)]], gsh, gsem[k]).wait()
        pltpu.make_async_copy(
            b_hbm.at[dh_v.at[pl.ds(off, n)]], gdh, gsem[k]).wait()

    def merge(k, n=CH2):
        gs, gd, gsh, gdh = gbufs[k]
        sb = sbufs[k]

        def rows(r2, _):
            for rr in range(2):
                r = r2 * 2 + rr
                for j in range(EH // 16):
                    sl16 = pl.ds(j * 16, 16)
                    sr16 = pl.ds(EH + j * 16, 16)
                    sb[r, sl16] = gs[r, sl16] + gd[r, sl16]
                    sb[r, sr16] = gsh[r, sl16] + gdh[r, sl16]
            return 0

        lax.fori_loop(0, n // 2, rows, 0)

    def wait_write(ci, k, n=CH2):
        pltpu.make_async_copy(
            bufs(k, n)[4], out_hbm.at[pl.ds(rbase + ci * CH2, n)],
            wsem[k]).wait()

    for k in range(PAIR_R):
        gat(k, k)

    nring = NFULL2 // PAIR_R

    def body(i, _):
        for k in range(PAIR_R):
            ci = i * PAIR_R + k
            wait_gat(ci, k)

            @pl.when(i > 0)
            def _():
                wait_write(ci, k)

            merge(k)
            pltpu.async_copy(
                sbufs[k], out_hbm.at[pl.ds(rbase + ci * CH2, CH2)], wsem[k])

            @pl.when(ci + PAIR_R < NFULL2)
            def _():
                gat(ci + PAIR_R, k)

        return 0

    lax.fori_loop(0, nring, body, 0)
    for k in range(NFULL2 - nring * PAIR_R):
        ci = nring * PAIR_R + k
        wait_gat(ci, k)
        wait_write(ci, k)
        merge(k)
        pltpu.sync_copy(sbufs[k], out_hbm.at[pl.ds(rbase + ci * CH2, CH2)])
    for k in range(NFULL2 - nring * PAIR_R, PAIR_R):
        wait_write(0, k)
    gat(NFULL2, 0, TAIL2)
    wait_gat(NFULL2, 0, TAIL2)
    merge(0, TAIL2)
    pltpu.sync_copy(bufs(0, TAIL2)[4],
                    out_hbm.at[pl.ds(rbase + NFULL2 * CH2, TAIL2)])


@functools.cache
def _sc_kernels():
    mesh = plsc.VectorSubcoreMesh(
        core_axis_name="c", subcore_axis_name="s",
        num_cores=NC, num_subcores=NS)
    sem = pltpu.SemaphoreType.DMA
    deg = pl.kernel(
        _deg_body,
        out_type=jax.ShapeDtypeStruct((NC, N), jnp.float32),
        mesh=mesh,
        scratch_types=[
            pltpu.VMEM((NCHUNK, CH), jnp.int32),
            pltpu.VMEM((CH,), jnp.float32),
            pltpu.VMEM_SHARED((N,), jnp.float32),
        ] + [sem] * DEG_R,
    )
    agg = pl.kernel(
        _agg_body,
        out_type=jax.ShapeDtypeStruct((NC, N, D), jnp.float32),
        mesh=mesh,
        scratch_types=[
            pltpu.VMEM((EPT,), jnp.int32),
            pltpu.VMEM((NCHUNK, CH), jnp.int32),
        ] + [pltpu.VMEM((CH, D), jnp.float32)] * AGG_R + [
            pltpu.VMEM_SHARED((N, D), jnp.float32),
        ] + [sem] * (2 * AGG_R),
    )
    pair = pl.kernel(
        _pair_body,
        out_type=jax.ShapeDtypeStruct((EHALF, 2 * EH), jnp.float32),
        mesh=mesh,
        scratch_types=(
            [pltpu.VMEM((RPT,), jnp.int32)] * 4
            + [pltpu.VMEM((CH2, EH), jnp.float32)] * (4 * PAIR_R)
            + [pltpu.VMEM((CH2, 2 * EH), jnp.float32)] * PAIR_R
            + [sem] * (2 * PAIR_R)),
        compiler_params=pltpu.CompilerParams(use_tc_tiling_on_sc=False),
    )
    return deg, agg, pair


BM = 2000
BM4 = 3200


def _tc1_body(x_ref, w_ref, degt_ref, xlin_ref, y1_ref, dinv_ref):
    dinv = lax.rsqrt(degt_ref[:, 0:1] + degt_ref[:, 1:2] + 1.0)
    xl = jnp.dot(x_ref[...], w_ref[...], preferred_element_type=jnp.float32)
    xlin_ref[...] = xl
    y1_ref[...] = xl * dinv
    dinv_ref[...] = dinv


def _tc2_body(agg_ref, xlin_ref, dinv_ref, b1_ref, w2_ref, hlin_ref, y2_ref):
    dinv = dinv_ref[...]
    aggsum = agg_ref[0] + agg_ref[1]
    h1 = jnp.maximum(
        dinv * aggsum + (dinv * dinv) * xlin_ref[...] + b1_ref[...], 0.0)
    hl = jnp.dot(h1, w2_ref[...], preferred_element_type=jnp.float32)
    hlin_ref[...] = hl
    y2_ref[...] = hl * dinv


def _tc3_body(agg_ref, hlin_ref, dinv_ref, b2_ref, we1_ref, a_ref, b_ref):
    dinv = dinv_ref[...]
    aggsum = agg_ref[0] + agg_ref[1]
    h = dinv * aggsum + (dinv * dinv) * hlin_ref[...] + b2_ref[...]
    we1 = we1_ref[...]
    a_ref[...] = jnp.dot(h, we1[:D], preferred_element_type=jnp.float32)
    b_ref[...] = jnp.dot(h, we1[D:], preferred_element_type=jnp.float32)


def _softmax_t(lt):
    m = jnp.max(lt, axis=0, keepdims=True)
    ex = jnp.exp(lt - m)
    return ex / jnp.sum(ex, axis=0, keepdims=True)


def _tc4_body(s2_ref, be1_ref, we2t_ref, be2t_ref,
              ltl_ref, lth_ref, ptl_ref, pth_ref, kl_ref, rc_ref, loss_ref):
    pi = pl.program_id(0)
    s2 = s2_ref[...]
    we2t = we2t_ref[...]
    be2t = be2t_ref[...]
    plp = jnp.log(jnp.float32(1.0 / 3.0) + jnp.float32(1e-12))

    kl = jnp.float32(0.0)
    rc = jnp.float32(0.0)
    for half, (l_ref, p_ref) in enumerate(((ltl_ref, ptl_ref),
                                           (lth_ref, pth_ref))):
        shalf = s2[:, half * EH:(half + 1) * EH].astype(jnp.float32)
        hid = jnp.maximum(shalf + be1_ref[...], 0.0)
        lt = lax.dot_general(we2t, hid, (((1,), (1,)), ((), ())),
                             preferred_element_type=jnp.float32) + be2t
        pt = _softmax_t(lt)
        l_ref[...] = lt
        p_ref[...] = pt
        kl += jnp.sum(pt * (jnp.log(jnp.maximum(pt, 1e-12)) - plp))
        rc += jnp.sum(jnp.log(jnp.maximum(pt[0:1, :] + pt[2:3, :], 1e-12)))

    @pl.when(pi == 0)
    def _():
        kl_ref[...] = jnp.zeros((1, 1), jnp.float32)
        rc_ref[...] = jnp.zeros((1, 1), jnp.float32)

    kl_ref[...] += kl.reshape(1, 1)
    rc_ref[...] += rc.reshape(1, 1)

    @pl.when(pi == (EHALF // BM4) - 1)
    def _():
        loss_ref[...] = (kl_ref[...] - rc_ref[...]) * jnp.float32(1.0 / E)


def _row_spec(bm, cols):
    return pl.BlockSpec((bm, cols), lambda i: (i, 0))


def _whole_spec(shape):
    return pl.BlockSpec(shape, lambda i: tuple(0 for _ in shape))


def _tc1(x, w1, degt):
    return pl.pallas_call(
        _tc1_body,
        grid=(N // BM,),
        in_specs=[_row_spec(BM, D), _whole_spec((D, D)), _row_spec(BM, 2)],
        out_specs=[_row_spec(BM, D), _row_spec(BM, D), _row_spec(BM, 1)],
        out_shape=[
            jax.ShapeDtypeStruct((N, D), jnp.float32),
            jax.ShapeDtypeStruct((N, D), jnp.float32),
            jax.ShapeDtypeStruct((N, 1), jnp.float32),
        ],
    )(x, w1, degt)


def _tc2(agg, xlin, dinv, b1, w2):
    return pl.pallas_call(
        _tc2_body,
        grid=(N // BM,),
        in_specs=[
            pl.BlockSpec((NC, BM, D), lambda i: (0, i, 0)),
            _row_spec(BM, D), _row_spec(BM, 1),
            _whole_spec((1, D)), _whole_spec((D, D)),
        ],
        out_specs=[_row_spec(BM, D), _row_spec(BM, D)],
        out_shape=[
            jax.ShapeDtypeStruct((N, D), jnp.float32),
            jax.ShapeDtypeStruct((N, D), jnp.float32),
        ],
    )(agg, xlin, dinv, b1, w2)


def _tc3(agg, hlin, dinv, b2, we1):
    return pl.pallas_call(
        _tc3_body,
        grid=(N // BM,),
        in_specs=[
            pl.BlockSpec((NC, BM, D), lambda i: (0, i, 0)),
            _row_spec(BM, D), _row_spec(BM, 1),
            _whole_spec((1, D)), _whole_spec((2 * D, EH)),
        ],
        out_specs=[_row_spec(BM, EH), _row_spec(BM, EH)],
        out_shape=[
            jax.ShapeDtypeStruct((N, EH), jnp.float32),
            jax.ShapeDtypeStruct((N, EH), jnp.float32),
        ],
    )(agg, hlin, dinv, b2, we1)


def _tc4(spack, be1, we2t, be2t):
    tspec = pl.BlockSpec((3, BM4), lambda i: (0, i))
    tshape = jax.ShapeDtypeStruct((3, EHALF), jnp.float32)
    sshape = jax.ShapeDtypeStruct((1, 1), jnp.float32)
    return pl.pallas_call(
        _tc4_body,
        grid=(EHALF // BM4,),
        in_specs=[
            _row_spec(BM4, 2 * EH), _whole_spec((1, EH)),
            _whole_spec((3, EH)), _whole_spec((3, 1)),
        ],
        out_specs=[
            tspec, tspec, tspec, tspec,
            _whole_spec((1, 1)), _whole_spec((1, 1)), _whole_spec((1, 1)),
        ],
        out_shape=[
            tshape, tshape, tshape, tshape, sshape, sshape, sshape,
        ],
    )(spack, be1, we2t, be2t)


@jax.jit
def kernel(x, edge_index, W1, b1, W2, b2, We1, be1, We2, be2):
    _deg, _agg, _pair = _sc_kernels()
    src = edge_index[0]
    dst = edge_index[1]
    dst3 = dst.reshape(NC * NS, NCHUNK, CH)
    z1 = jnp.zeros((N,), jnp.float32)
    z2 = jnp.zeros((N, D), jnp.float32)

    degp = _deg(dst3, z1)
    degt = degp.T

    xlin, y1, dinv = _tc1(x, W1, degt)
    agg1 = _agg(y1, src, dst3, z2)
    hlin, y2 = _tc2(agg1, xlin, dinv, b1.reshape(1, D), W2)
    agg2 = _agg(y2, src, dst3, z2)
    a, b = _tc3(agg2, hlin, dinv, b2.reshape(1, D), We1)

    spack = _pair(a, b, src, dst)

    ltl, lth, ptl, pth, _, _, loss = _tc4(
        spack, be1.reshape(1, EH), We2.T, be2.reshape(3, 1))
    logits = jnp.concatenate([ltl.T, lth.T], axis=0)
    probs = jnp.concatenate([ptl.T, pth.T], axis=0)
    return logits, probs, loss.reshape(())


# speedup vs baseline: 1.1159x; 1.1159x over previous
import functools

import jax
import jax.numpy as jnp
from jax import lax
from jax.experimental import pallas as pl
from jax.experimental.pallas import tpu as pltpu
from jax.experimental.pallas import tpu_sc as plsc

N = 10000
E = 320000
D = 128
EH = 64

NC = 2
NS = 16
EPC = E // NC
EPT = EPC // NS
CH = 80
NCHUNK = EPT // CH

ROWS_A = 640
ROWS_B = N - 15 * ROWS_A


NPAIR = NCHUNK // 2


DEG_R = 4
AGG_R = 3


def _deg_body(dst3_hbm, zeros_hbm, out_hbm, didx_v, ones_v, acc, *sems):
    c = lax.axis_index("c")
    s = lax.axis_index("s")
    w = c * NS + s

    @pl.when(s == 0)
    def _():
        pltpu.sync_copy(zeros_hbm, acc)

    pltpu.sync_copy(dst3_hbm.at[w], didx_v)
    for i in range(CH // 16):
        ones_v[pl.ds(i * 16, 16)] = jnp.ones((16,), jnp.float32)
    plsc.subcore_barrier()

    def scat(ci, k):
        pltpu.async_copy(ones_v, acc.at[didx_v.at[ci]], sems[k], add=True)

    def wait(ci, k):
        pltpu.make_async_copy(ones_v, acc.at[didx_v.at[ci]], sems[k]).wait()

    for k in range(DEG_R):
        scat(k, k)

    nfull = (NCHUNK - 1) // DEG_R

    def body(i, _):
        for k in range(DEG_R):
            ci = i * DEG_R + k
            wait(ci, k)

            @pl.when(ci + DEG_R < NCHUNK)
            def _():
                scat(ci + DEG_R, k)

        return 0

    lax.fori_loop(0, nfull, body, 0)
    for k in range(NCHUNK - nfull * DEG_R):
        wait(nfull * DEG_R + k, k)
    plsc.subcore_barrier()

    @pl.when(s == 0)
    def _():
        pltpu.sync_copy(acc, out_hbm.at[c])


def _agg_body(y_hbm, src_hbm, dst3_hbm, zeros_hbm, out_hbm,
              didx_v, *rest):
    sidx = rest[:AGG_R]
    rows = rest[AGG_R:2 * AGG_R]
    acc = rest[2 * AGG_R]
    gsem = rest[2 * AGG_R + 1:3 * AGG_R + 1]
    ssem = rest[3 * AGG_R + 1:]
    c = lax.axis_index("c")
    s = lax.axis_index("s")
    w = c * NS + s

    @pl.when(s < 15)
    def _():
        pltpu.sync_copy(zeros_hbm.at[pl.ds(s * ROWS_A, ROWS_A)],
                        acc.at[pl.ds(s * ROWS_A, ROWS_A)])

    @pl.when(s == 15)
    def _():
        pltpu.sync_copy(zeros_hbm.at[pl.ds(15 * ROWS_A, ROWS_B)],
                        acc.at[pl.ds(15 * ROWS_A, ROWS_B)])

    pltpu.sync_copy(dst3_hbm.at[w], didx_v)
    plsc.subcore_barrier()

    def gat(ci, k):
        pltpu.sync_copy(src_hbm.at[pl.ds(w * EPT + ci * CH, CH)], sidx[k])
        pltpu.async_copy(y_hbm.at[sidx[k]], rows[k], gsem[k])

    def wait_gat(ci, k):
        pltpu.make_async_copy(y_hbm.at[sidx[k]], rows[k], gsem[k]).wait()

    def scat(ci, k):
        pltpu.async_copy(rows[k], acc.at[didx_v.at[ci]], ssem[k], add=True)

    def wait_scat(ci, k):
        pltpu.make_async_copy(rows[k], acc.at[didx_v.at[ci]], ssem[k]).wait()

    for k in range(AGG_R):
        gat(k, k)

    nfull = NCHUNK // AGG_R

    def body(i, _):
        for k in range(AGG_R):
            ci = i * AGG_R + k
            wait_gat(ci, k)
            scat(ci, k)
        for k in range(AGG_R):
            ci = i * AGG_R + k
            wait_scat(ci, k)

            @pl.when(ci + AGG_R < NCHUNK)
            def _():
                gat(ci + AGG_R, k)

        return 0

    lax.fori_loop(0, nfull, body, 0)
    for k in range(NCHUNK - nfull * AGG_R):
        ci = nfull * AGG_R + k
        wait_gat(ci, k)
        scat(ci, k)
    for k in range(NCHUNK - nfull * AGG_R):
        wait_scat(nfull * AGG_R + k, k)
    plsc.subcore_barrier()

    @pl.when(s < 15)
    def _():
        pltpu.sync_copy(acc.at[pl.ds(s * ROWS_A, ROWS_A)],
                        out_hbm.at[c, pl.ds(s * ROWS_A, ROWS_A)])

    @pl.when(s == 15)
    def _():
        pltpu.sync_copy(acc.at[pl.ds(15 * ROWS_A, ROWS_B)],
                        out_hbm.at[c, pl.ds(15 * ROWS_A, ROWS_B)])


EHALF = E // 2
RPT = EHALF // (NC * NS)
CH2 = 80
NFULL2 = RPT // CH2
TAIL2 = RPT - NFULL2 * CH2


PAIR_R = 3


def _pair_body(a_hbm, b_hbm, src_hbm, dst_hbm, out_hbm,
               sl_v, dl_v, sh_v, dh_v, *rest):
    gbufs = [rest[4 * k:4 * k + 4] for k in range(PAIR_R)]
    sbufs = rest[4 * PAIR_R:5 * PAIR_R]
    gsem = rest[5 * PAIR_R:6 * PAIR_R]
    wsem = rest[6 * PAIR_R:7 * PAIR_R]
    c = lax.axis_index("c")
    s = lax.axis_index("s")
    w = c * NS + s
    rbase = w * RPT

    pltpu.sync_copy(src_hbm.at[pl.ds(rbase, RPT)], sl_v)
    pltpu.sync_copy(dst_hbm.at[pl.ds(rbase, RPT)], dl_v)
    pltpu.sync_copy(src_hbm.at[pl.ds(EHALF + rbase, RPT)], sh_v)
    pltpu.sync_copy(dst_hbm.at[pl.ds(EHALF + rbase, RPT)], dh_v)

    def bufs(k, n):
        gs, gd, gsh, gdh = gbufs[k]
        if n == CH2:
            return gs, gd, gsh, gdh, sbufs[k]
        sl = pl.ds(0, n)
        return (gs.at[sl], gd.at[sl], gsh.at[sl], gdh.at[sl],
                sbufs[k].at[sl])

    def gat(ci, k, n=CH2):
        gs, gd, gsh, gdh, _ = bufs(k, n)
        off = ci * CH2
        pltpu.async_copy(a_hbm.at[sl_v.at[pl.ds(off, n)]], gs, gsem[k])
        pltpu.async_copy(b_hbm.at[dl_v.at[pl.ds(off, n)]], gd, gsem[k])
        pltpu.async_copy(a_hbm.at[sh_v.at[pl.ds(off, n)]], gsh, gsem[k])
        pltpu.async_copy(b_hbm.at[dh_v.at[pl.ds(off, n)]], gdh, gsem[k])

    def wait_gat(ci, k, n=CH2):
        gs, gd, gsh, gdh, _ = bufs(k, n)
        off = ci * CH2
        pltpu.make_async_copy(
            a_hbm.at[sl_v.at[pl.ds(off, n)]], gs, gsem[k]).wait()
        pltpu.make_async_copy(
            b_hbm.at[dl_v.at[pl.ds(off, n)]], gd, gsem[k]).wait()
        pltpu.make_async_copy(
            a_hbm.at[sh_v.at[pl.ds(off, n)]], gsh, gsem[k]).wait()
        pltpu.make_async_copy(
            b_hbm.at[dh_v.at[pl.ds(off, n)]], gdh, gsem[k]).wait()

    def merge(k, n=CH2):
        gs, gd, gsh, gdh = gbufs[k]
        sb = sbufs[k]

        def rows(r2, _):
            for rr in range(2):
                r = r2 * 2 + rr
                for j in range(EH // 16):
                    sl16 = pl.ds(j * 16, 16)
                    sr16 = pl.ds(EH + j * 16, 16)
                    sb[r, sl16] = gs[r, sl16] + gd[r, sl16]
                    sb[r, sr16] = gsh[r, sl16] + gdh[r, sl16]
            return 0

        lax.fori_loop(0, n // 2, rows, 0)

    def wait_write(ci, k, n=CH2):
        pltpu.make_async_copy(
            bufs(k, n)[4], out_hbm.at[pl.ds(rbase + ci * CH2, n)],
            wsem[k]).wait()

    for k in range(PAIR_R):
        gat(k, k)

    nring = NFULL2 // PAIR_R

    def body(i, _):
        for k in range(PAIR_R):
            ci = i * PAIR_R + k
            wait_gat(ci, k)

            @pl.when(i > 0)
            def _():
                wait_write(ci, k)

            merge(k)
            pltpu.async_copy(
                sbufs[k], out_hbm.at[pl.ds(rbase + ci * CH2, CH2)], wsem[k])

            @pl.when(ci + PAIR_R < NFULL2)
            def _():
                gat(ci + PAIR_R, k)

        return 0

    lax.fori_loop(0, nring, body, 0)
    for k in range(NFULL2 - nring * PAIR_R):
        ci = nring * PAIR_R + k
        wait_gat(ci, k)
        wait_write(ci, k)
        merge(k)
        pltpu.sync_copy(sbufs[k], out_hbm.at[pl.ds(rbase + ci * CH2, CH2)])
    for k in range(NFULL2 - nring * PAIR_R, PAIR_R):
        wait_write(0, k)
    gat(NFULL2, 0, TAIL2)
    wait_gat(NFULL2, 0, TAIL2)
    merge(0, TAIL2)
    pltpu.sync_copy(bufs(0, TAIL2)[4],
                    out_hbm.at[pl.ds(rbase + NFULL2 * CH2, TAIL2)])


@functools.cache
def _sc_kernels():
    mesh = plsc.VectorSubcoreMesh(
        core_axis_name="c", subcore_axis_name="s",
        num_cores=NC, num_subcores=NS)
    sem = pltpu.SemaphoreType.DMA
    deg = pl.kernel(
        _deg_body,
        out_type=jax.ShapeDtypeStruct((NC, N), jnp.float32),
        mesh=mesh,
        scratch_types=[
            pltpu.VMEM((NCHUNK, CH), jnp.int32),
            pltpu.VMEM((CH,), jnp.float32),
            pltpu.VMEM_SHARED((N,), jnp.float32),
        ] + [sem] * DEG_R,
    )
    agg = pl.kernel(
        _agg_body,
        out_type=jax.ShapeDtypeStruct((NC, N, D), jnp.float32),
        mesh=mesh,
        scratch_types=[
            pltpu.VMEM((NCHUNK, CH), jnp.int32),
        ] + [pltpu.VMEM((CH,), jnp.int32)] * AGG_R
        + [pltpu.VMEM((CH, D), jnp.float32)] * AGG_R + [
            pltpu.VMEM_SHARED((N, D), jnp.float32),
        ] + [sem] * (2 * AGG_R),
    )
    pair = pl.kernel(
        _pair_body,
        out_type=jax.ShapeDtypeStruct((EHALF, 2 * EH), jnp.float32),
        mesh=mesh,
        scratch_types=(
            [pltpu.VMEM((RPT,), jnp.int32)] * 4
            + [pltpu.VMEM((CH2, EH), jnp.float32)] * (4 * PAIR_R)
            + [pltpu.VMEM((CH2, 2 * EH), jnp.float32)] * PAIR_R
            + [sem] * (2 * PAIR_R)),
        compiler_params=pltpu.CompilerParams(use_tc_tiling_on_sc=False),
    )
    return deg, agg, pair


BM = 2000
BM4 = 6400


def _tc1_body(x_ref, w_ref, degt_ref, xlin_ref, y1_ref, dinv_ref):
    dinv = lax.rsqrt(degt_ref[:, 0:1] + degt_ref[:, 1:2] + 1.0)
    xl = jnp.dot(x_ref[...], w_ref[...], preferred_element_type=jnp.float32)
    xlin_ref[...] = xl
    y1_ref[...] = xl * dinv
    dinv_ref[...] = dinv


def _tc2_body(agg_ref, xlin_ref, dinv_ref, b1_ref, w2_ref, hlin_ref, y2_ref):
    dinv = dinv_ref[...]
    aggsum = agg_ref[0] + agg_ref[1]
    h1 = jnp.maximum(
        dinv * aggsum + (dinv * dinv) * xlin_ref[...] + b1_ref[...], 0.0)
    hl = jnp.dot(h1, w2_ref[...], preferred_element_type=jnp.float32)
    hlin_ref[...] = hl
    y2_ref[...] = hl * dinv


def _tc3_body(agg_ref, hlin_ref, dinv_ref, b2_ref, we1_ref, a_ref, b_ref):
    dinv = dinv_ref[...]
    aggsum = agg_ref[0] + agg_ref[1]
    h = dinv * aggsum + (dinv * dinv) * hlin_ref[...] + b2_ref[...]
    we1 = we1_ref[...]
    a_ref[...] = jnp.dot(h, we1[:D], preferred_element_type=jnp.float32)
    b_ref[...] = jnp.dot(h, we1[D:], preferred_element_type=jnp.float32)


def _softmax_t(lt):
    m = jnp.max(lt, axis=0, keepdims=True)
    ex = jnp.exp(lt - m)
    return ex / jnp.sum(ex, axis=0, keepdims=True)


def _tc4_body(s2_ref, be1_ref, we2t_ref, be2t_ref,
              ltl_ref, lth_ref, ptl_ref, pth_ref, kl_ref, rc_ref, loss_ref):
    pi = pl.program_id(0)
    s2 = s2_ref[...]
    we2t = we2t_ref[...]
    be2t = be2t_ref[...]
    plp = jnp.log(jnp.float32(1.0 / 3.0) + jnp.float32(1e-12))

    kl = jnp.float32(0.0)
    rc = jnp.float32(0.0)
    for half, (l_ref, p_ref) in enumerate(((ltl_ref, ptl_ref),
                                           (lth_ref, pth_ref))):
        shalf = s2[:, half * EH:(half + 1) * EH].astype(jnp.float32)
        hid = jnp.maximum(shalf + be1_ref[...], 0.0)
        lt = lax.dot_general(we2t, hid, (((1,), (1,)), ((), ())),
                             preferred_element_type=jnp.float32) + be2t
        pt = _softmax_t(lt)
        l_ref[...] = lt
        p_ref[...] = pt
        kl += jnp.sum(pt * (jnp.log(jnp.maximum(pt, 1e-12)) - plp))
        rc += jnp.sum(jnp.log(jnp.maximum(pt[0:1, :] + pt[2:3, :], 1e-12)))

    @pl.when(pi == 0)
    def _():
        kl_ref[...] = jnp.zeros((1, 1), jnp.float32)
        rc_ref[...] = jnp.zeros((1, 1), jnp.float32)

    kl_ref[...] += kl.reshape(1, 1)
    rc_ref[...] += rc.reshape(1, 1)

    @pl.when(pi == (EHALF // BM4) - 1)
    def _():
        loss_ref[...] = (kl_ref[...] - rc_ref[...]) * jnp.float32(1.0 / E)


def _row_spec(bm, cols):
    return pl.BlockSpec((bm, cols), lambda i: (i, 0))


def _whole_spec(shape):
    return pl.BlockSpec(shape, lambda i: tuple(0 for _ in shape))


def _tc1(x, w1, degt):
    return pl.pallas_call(
        _tc1_body,
        grid=(N // BM,),
        in_specs=[_row_spec(BM, D), _whole_spec((D, D)), _row_spec(BM, 2)],
        out_specs=[_row_spec(BM, D), _row_spec(BM, D), _row_spec(BM, 1)],
        out_shape=[
            jax.ShapeDtypeStruct((N, D), jnp.float32),
            jax.ShapeDtypeStruct((N, D), jnp.float32),
            jax.ShapeDtypeStruct((N, 1), jnp.float32),
        ],
    )(x, w1, degt)


def _tc2(agg, xlin, dinv, b1, w2):
    return pl.pallas_call(
        _tc2_body,
        grid=(N // BM,),
        in_specs=[
            pl.BlockSpec((NC, BM, D), lambda i: (0, i, 0)),
            _row_spec(BM, D), _row_spec(BM, 1),
            _whole_spec((1, D)), _whole_spec((D, D)),
        ],
        out_specs=[_row_spec(BM, D), _row_spec(BM, D)],
        out_shape=[
            jax.ShapeDtypeStruct((N, D), jnp.float32),
            jax.ShapeDtypeStruct((N, D), jnp.float32),
        ],
    )(agg, xlin, dinv, b1, w2)


def _tc3(agg, hlin, dinv, b2, we1):
    return pl.pallas_call(
        _tc3_body,
        grid=(N // BM,),
        in_specs=[
            pl.BlockSpec((NC, BM, D), lambda i: (0, i, 0)),
            _row_spec(BM, D), _row_spec(BM, 1),
            _whole_spec((1, D)), _whole_spec((2 * D, EH)),
        ],
        out_specs=[_row_spec(BM, EH), _row_spec(BM, EH)],
        out_shape=[
            jax.ShapeDtypeStruct((N, EH), jnp.float32),
            jax.ShapeDtypeStruct((N, EH), jnp.float32),
        ],
    )(agg, hlin, dinv, b2, we1)


def _tc4(spack, be1, we2t, be2t):
    tspec = pl.BlockSpec((3, BM4), lambda i: (0, i))
    tshape = jax.ShapeDtypeStruct((3, EHALF), jnp.float32)
    sshape = jax.ShapeDtypeStruct((1, 1), jnp.float32)
    return pl.pallas_call(
        _tc4_body,
        grid=(EHALF // BM4,),
        in_specs=[
            _row_spec(BM4, 2 * EH), _whole_spec((1, EH)),
            _whole_spec((3, EH)), _whole_spec((3, 1)),
        ],
        out_specs=[
            tspec, tspec, tspec, tspec,
            _whole_spec((1, 1)), _whole_spec((1, 1)), _whole_spec((1, 1)),
        ],
        out_shape=[
            tshape, tshape, tshape, tshape, sshape, sshape, sshape,
        ],
    )(spack, be1, we2t, be2t)


@jax.jit
def kernel(x, edge_index, W1, b1, W2, b2, We1, be1, We2, be2):
    _deg, _agg, _pair = _sc_kernels()
    src = edge_index[0]
    dst = edge_index[1]
    dst3 = dst.reshape(NC * NS, NCHUNK, CH)
    z1 = jnp.zeros((N,), jnp.float32)
    z2 = jnp.zeros((N, D), jnp.float32)

    degp = _deg(dst3, z1)
    degt = degp.T

    xlin, y1, dinv = _tc1(x, W1, degt)
    agg1 = _agg(y1, src, dst3, z2)
    hlin, y2 = _tc2(agg1, xlin, dinv, b1.reshape(1, D), W2)
    agg2 = _agg(y2, src, dst3, z2)
    a, b = _tc3(agg2, hlin, dinv, b2.reshape(1, D), We1)

    spack = _pair(a, b, src, dst)

    ltl, lth, ptl, pth, _, _, loss = _tc4(
        spack, be1.reshape(1, EH), We2.T, be2.reshape(3, 1))
    logits = jnp.concatenate([ltl.T, lth.T], axis=0)
    probs = jnp.concatenate([ptl.T, pth.T], axis=0)
    return logits, probs, loss.reshape(())
